# baseline (speedup 1.0000x reference)
import functools

import jax
import jax.numpy as jnp
from jax.experimental import pallas as pl
from jax.experimental.pallas import tpu as pltpu
from jax.experimental.pallas import tpu_sc as plsc

_K = 16


def _fps_body(n_samples, px_ref, py_ref, pz_ref, cx_ref, cy_ref, cz_ref,
              idx_ref, sx_ref, sy_ref, sz_ref, dist_ref):
    R, C = px_ref.shape
    iota = (jax.lax.broadcasted_iota(jnp.int32, (R, C), 0) * C
            + jax.lax.broadcasted_iota(jnp.int32, (R, C), 1))
    idx_ref[0:1, :] = jnp.zeros((1, 1), jnp.int32)
    sx_ref[0:1, :] = px_ref[0:1, 0:1]
    sy_ref[0:1, :] = py_ref[0:1, 0:1]
    sz_ref[0:1, :] = pz_ref[0:1, 0:1]
    dist_ref[...] = jnp.full((R, C), 1e18, jnp.float32)

    def body(i, carry):
        lx, ly, lz = carry
        dx = px_ref[...] - lx
        dy = py_ref[...] - ly
        dz = pz_ref[...] - lz
        d = dx * dx + dy * dy + dz * dz
        nd = jnp.minimum(dist_ref[...], d)
        dist_ref[...] = nd
        m = jnp.max(nd)
        cand = jnp.where(nd == m, iota, jnp.int32(2 ** 30))
        j = jnp.min(cand)
        idx_ref[pl.ds(i, 1), :] = jnp.reshape(j, (1, 1))
        nlx = cx_ref[j, 0]
        nly = cy_ref[j, 0]
        nlz = cz_ref[j, 0]
        sx_ref[pl.ds(i, 1), :] = jnp.reshape(nlx, (1, 1))
        sy_ref[pl.ds(i, 1), :] = jnp.reshape(nly, (1, 1))
        sz_ref[pl.ds(i, 1), :] = jnp.reshape(nlz, (1, 1))
        return nlx, nly, nlz

    jax.lax.fori_loop(1, n_samples, body,
                      (px_ref[0, 0], py_ref[0, 0], pz_ref[0, 0]))


def _fps_pallas(pos, n_samples, rows):
    n = pos.shape[0]
    cols = n // rows
    px = pos[:, 0].reshape(rows, cols)
    py = pos[:, 1].reshape(rows, cols)
    pz = pos[:, 2].reshape(rows, cols)
    out = pl.pallas_call(
        functools.partial(_fps_body, n_samples),
        out_shape=[jax.ShapeDtypeStruct((n_samples, 1), jnp.int32)]
        + [jax.ShapeDtypeStruct((n_samples, 1), jnp.float32)] * 3,
        scratch_shapes=[pltpu.VMEM((rows, cols), jnp.float32)],
    )(px, py, pz, pos[:, 0].reshape(n, 1), pos[:, 1].reshape(n, 1),
      pos[:, 2].reshape(n, 1))
    sel = jnp.concatenate([out[1], out[2], out[3]], axis=1)
    return out[0][:, 0], sel


def _knn_body(k, nb, self_exclude, q_ref, bt_ref, idx_ref, d2_ref):
    R = q_ref.shape[0]
    q = q_ref[...]
    bt = bt_ref[...]
    qq = jnp.sum(q * q, axis=1, keepdims=True)
    bb = jnp.sum(bt * bt, axis=0, keepdims=True)
    ab = jnp.dot(q, bt, preferred_element_type=jnp.float32)
    D = qq - 2.0 * ab + bb
    lane = jax.lax.broadcasted_iota(jnp.int32, (R, nb), 1)
    if self_exclude:
        row = (jax.lax.broadcasted_iota(jnp.int32, (R, nb), 0)
               + pl.program_id(0) * R)
        D = jnp.where(row == lane, 1e10, D)
    big = jnp.int32(2 ** 30)
    for e in range(k):
        m = jnp.min(D, axis=1, keepdims=True)
        j = jnp.min(jnp.where(D == m, lane, big), axis=1, keepdims=True)
        D = jnp.where(lane == j, 1e30, D)
        idx_ref[:, e:e + 1] = j
        d2_ref[:, e:e + 1] = jnp.maximum(m, 0.0)


def _knn_pallas(query, base, k, self_exclude=False, block_r=512):
    nq = query.shape[0]
    nb = base.shape[0]
    r = min(block_r, nq)
    bt = base.T
    idx, d2 = pl.pallas_call(
        functools.partial(_knn_body, k, nb, self_exclude),
        grid=(nq // r,),
        in_specs=[
            pl.BlockSpec((r, 3), lambda i: (i, 0)),
            pl.BlockSpec((3, nb), lambda i: (0, 0)),
        ],
        out_specs=[
            pl.BlockSpec((r, k), lambda i: (i, 0)),
            pl.BlockSpec((r, k), lambda i: (i, 0)),
        ],
        out_shape=[
            jax.ShapeDtypeStruct((nq, k), jnp.int32),
            jax.ShapeDtypeStruct((nq, k), jnp.float32),
        ],
    )(query, bt)
    return d2, idx


def _sc_gather_rows(table, idx):
    V, D = table.shape
    B = idx.shape[0]
    NW = 32
    C = B // (NW * 128)
    idx3 = idx.reshape(NW, C, 128)
    mesh = plsc.VectorSubcoreMesh(core_axis_name="c", subcore_axis_name="s")

    @functools.partial(
        pl.kernel, mesh=mesh,
        out_type=jax.ShapeDtypeStruct((B, D), jnp.float32),
        scratch_types=[
            pltpu.VMEM((128,), jnp.int32),
            pltpu.VMEM((128, D), jnp.float32),
            pltpu.SemaphoreType.DMA,
        ],
    )
    def k(table_hbm, idx_hbm, out_hbm, idx_v, rows_v, sem):
        wid = jax.lax.axis_index("s") * 2 + jax.lax.axis_index("c")

        def chunk(c, carry):
            pltpu.sync_copy(idx_hbm.at[wid, c], idx_v)
            pltpu.async_copy(table_hbm.at[idx_v], rows_v, sem).wait()
            base = (wid * C + c) * 128
            pltpu.sync_copy(rows_v, out_hbm.at[pl.ds(base, 128)])
            return carry

        jax.lax.fori_loop(0, C, chunk, 0)

    return k(table, idx3)


def _gather_rows(table, idx):
    v, d = table.shape
    b = idx.shape[0]
    dpad = (d + 127) // 128 * 128
    bpad = (b + 4095) // 4096 * 4096
    if dpad != d:
        table = jnp.pad(table, ((0, 0), (0, dpad - d)))
    if bpad != b:
        idx = jnp.pad(idx, (0, bpad - b))
    out = _sc_gather_rows(table, idx)
    return out[:b, :d]


def _mlp_bn_body(x_ref, w_ref, b_ref, g_ref, be_ref, o_ref):
    y = jnp.dot(x_ref[...], w_ref[...], preferred_element_type=jnp.float32)
    y = y + b_ref[...]
    mu = jnp.mean(y, axis=0, keepdims=True)
    var = jnp.mean((y - mu) * (y - mu), axis=0, keepdims=True)
    y = (y - mu) / jnp.sqrt(var + 1e-5) * g_ref[...] + be_ref[...]
    o_ref[...] = jnp.maximum(y, 0.0)


def _mlp_bn(p, x):
    n = x.shape[0]
    dout = p["W"].shape[1]
    return pl.pallas_call(
        _mlp_bn_body,
        out_shape=jax.ShapeDtypeStruct((n, dout), jnp.float32),
    )(x, p["W"], p["b"].reshape(1, -1), p["g"].reshape(1, -1),
      p["be"].reshape(1, -1))


def _maxpool_body(g_ref, o_ref):
    o_ref[...] = jnp.max(g_ref[...], axis=1)


def _maxpool_k(g3, bn=256):
    n, k, d = g3.shape
    bn = min(bn, n)
    return pl.pallas_call(
        _maxpool_body,
        grid=(n // bn,),
        in_specs=[pl.BlockSpec((bn, k, d), lambda i: (i, 0, 0))],
        out_specs=pl.BlockSpec((bn, d), lambda i: (i, 0)),
        out_shape=jax.ShapeDtypeStruct((n, d), jnp.float32),
    )(g3)


def _tf_pre_body(d, x_ref, pos_ref, wi_ref, bi_ref, wd_ref, bd_ref, ws_ref,
                 bs_ref, wv_ref, bv_ref, tab_ref, q_ref):
    h = jnp.dot(x_ref[...], wi_ref[...], preferred_element_type=jnp.float32)
    h = jnp.maximum(h + bi_ref[...], 0.0)
    q_ref[...] = jnp.dot(h, wd_ref[...],
                         preferred_element_type=jnp.float32) + bd_ref[...]
    tab_ref[...] = jnp.zeros(tab_ref.shape, jnp.float32)
    tab_ref[:, 0:d] = jnp.dot(h, ws_ref[...],
                              preferred_element_type=jnp.float32) + bs_ref[...]
    tab_ref[:, d:2 * d] = jnp.dot(h, wv_ref[...],
                                  preferred_element_type=jnp.float32) + bv_ref[...]
    tab_ref[:, 2 * d:2 * d + 3] = pos_ref[...]


def _tf_attn_body(d, g_ref, q_ref, pos_ref, w1_ref, b1_ref, w2_ref, b2_ref,
                  wa1_ref, ba1_ref, wa2_ref, ba2_ref, wo_ref, bo_ref, o_ref):
    bn, k, dp = g_ref.shape
    g3 = g_ref[...]
    pq3 = pos_ref[...].reshape(bn, 1, 3)
    rel3 = pq3 - g3[:, :, 2 * d:2 * d + 3]
    rel2 = rel3.reshape(bn * k, 3)
    t = jnp.dot(rel2, w1_ref[...], preferred_element_type=jnp.float32)
    t = jnp.maximum(t + b1_ref[...], 0.0)
    delta2 = jnp.dot(t, w2_ref[...],
                     preferred_element_type=jnp.float32) + b2_ref[...]
    delta3 = delta2.reshape(bn, k, d)
    q3 = q_ref[...].reshape(bn, 1, d)
    alpha3 = q3 - g3[:, :, 0:d] + delta3
    a2 = alpha3.reshape(bn * k, d)
    t2 = jnp.dot(a2, wa1_ref[...], preferred_element_type=jnp.float32)
    t2 = jnp.maximum(t2 + ba1_ref[...], 0.0)
    a2 = jnp.dot(t2, wa2_ref[...],
                 preferred_element_type=jnp.float32) + ba2_ref[...]
    alpha3 = a2.reshape(bn, k, d)
    m = jnp.max(alpha3, axis=1, keepdims=True)
    e = jnp.exp(alpha3 - m)
    p = e / jnp.sum(e, axis=1, keepdims=True)
    out = jnp.sum(p * (g3[:, :, d:2 * d] + delta3), axis=1)
    y = jnp.dot(out, wo_ref[...], preferred_element_type=jnp.float32)
    o_ref[...] = jnp.maximum(y + bo_ref[...], 0.0)


def _transformer_block(p, x, pos, nbr):
    n, d = x.shape
    dpad = (2 * d + 3 + 127) // 128 * 128
    tab, q = pl.pallas_call(
        functools.partial(_tf_pre_body, d),
        out_shape=[
            jax.ShapeDtypeStruct((n, dpad), jnp.float32),
            jax.ShapeDtypeStruct((n, d), jnp.float32),
        ],
    )(x, pos, p["lin_in"]["W"], p["lin_in"]["b"].reshape(1, -1),
      p["lin_dst"]["W"], p["lin_dst"]["b"].reshape(1, -1),
      p["lin_src"]["W"], p["lin_src"]["b"].reshape(1, -1),
      p["lin"]["W"], p["lin"]["b"].reshape(1, -1))
    g = _gather_rows(tab, nbr.reshape(-1))
    g3 = g.reshape(n, _K, dpad)
    bn = 256 if n >= 2048 else 128
    return pl.pallas_call(
        functools.partial(_tf_attn_body, d),
        grid=(n // bn,),
        in_specs=[
            pl.BlockSpec((bn, _K, dpad), lambda i: (i, 0, 0)),
            pl.BlockSpec((bn, d), lambda i: (i, 0)),
            pl.BlockSpec((bn, 3), lambda i: (i, 0)),
        ] + [pl.BlockSpec(None, lambda i: (0, 0))] * 10,
        out_specs=pl.BlockSpec((bn, d), lambda i: (i, 0)),
        out_shape=jax.ShapeDtypeStruct((n, d), jnp.float32),
    )(g3, q, pos,
      p["pos1"]["W"], p["pos1"]["b"].reshape(1, -1),
      p["pos2"]["W"], p["pos2"]["b"].reshape(1, -1),
      p["attn1"]["W"], p["attn1"]["b"].reshape(1, -1),
      p["attn2"]["W"], p["attn2"]["b"].reshape(1, -1),
      p["lin_out"]["W"], p["lin_out"]["b"].reshape(1, -1))


def _interp_body(x_ref, d2_ref, f0_ref, f1_ref, f2_ref, o_ref):
    w = 1.0 / jnp.maximum(d2_ref[...], 1e-16)
    w = w / jnp.sum(w, axis=1, keepdims=True)
    o_ref[...] = (x_ref[...] + w[:, 0:1] * f0_ref[...]
                  + w[:, 1:2] * f1_ref[...] + w[:, 2:3] * f2_ref[...])


def _interp_add(x_mlp, feat, d2, idx):
    n = x_mlp.shape[0]
    d = feat.shape[1]
    f = [_gather_rows(feat, idx[:, j]) for j in range(3)]
    return pl.pallas_call(
        _interp_body,
        out_shape=jax.ShapeDtypeStruct((n, d), jnp.float32),
    )(x_mlp, d2, f[0], f[1], f[2])


def _head_body(h_ref, w1_ref, b1_ref, w2_ref, b2_ref, w3_ref, b3_ref, o_ref):
    o = jnp.dot(h_ref[...], w1_ref[...], preferred_element_type=jnp.float32)
    o = jnp.maximum(o + b1_ref[...], 0.0)
    o = jnp.dot(o, w2_ref[...], preferred_element_type=jnp.float32)
    o = jnp.maximum(o + b2_ref[...], 0.0)
    o = jnp.dot(o, w3_ref[...], preferred_element_type=jnp.float32)
    o_ref[...] = o + b3_ref[...]


def _summit_body(h_ref, w_ref, b_ref, o_ref):
    o = jnp.dot(h_ref[...], w_ref[...], preferred_element_type=jnp.float32)
    o_ref[...] = jnp.maximum(o + b_ref[...], 0.0)


def _forward(x, pos, params, fps_idx, sub_pos):
    h = _mlp_bn(params["mlp_input"], x)
    out_x = [h]
    out_pos = [pos]
    for i in range(2):
        p_prev = out_pos[-1]
        sp = sub_pos[i]
        _, nbr = _knn_pallas(sp, p_prev, _K)
        hh = _mlp_bn(params["td"][i], out_x[-1])
        gh = _gather_rows(hh, nbr.reshape(-1))
        h = _maxpool_k(gh.reshape(sp.shape[0], _K, hh.shape[1]))
        nbr_g = _knn_pallas(sp, sp, _K, self_exclude=True)[1]
        h = _transformer_block(params["tf_down"][i], h, sp, nbr_g)
        out_x.append(h)
        out_pos.append(sp)
    n2, d2dim = h.shape
    h = pl.pallas_call(
        _summit_body,
        out_shape=jax.ShapeDtypeStruct((n2, d2dim), jnp.float32),
    )(h, params["mlp_summit"]["W"], params["mlp_summit"]["b"].reshape(1, -1))
    h = _transformer_block(params["tf_summit"], h, out_pos[-1],
                           _knn_pallas(out_pos[-1], out_pos[-1], _K,
                                       self_exclude=True)[1])
    for i in range(2):
        pu = params["tu"][-i - 1]
        x_skip = out_x[-i - 2]
        pos_up = out_pos[-i - 2]
        pos_sub = out_pos[-i - 1]
        h_sub = _mlp_bn(pu["mlp_sub"], h)
        d2i, idxi = _knn_pallas(pos_up, pos_sub, 3)
        x_mlp = _mlp_bn(pu["mlp"], x_skip)
        h = _interp_add(x_mlp, h_sub, d2i, idxi)
        h = _transformer_block(params["tf_up"][-i - 1], h, pos_up,
                               _knn_pallas(pos_up, pos_up, _K,
                                           self_exclude=True)[1])
    n0 = h.shape[0]
    return pl.pallas_call(
        _head_body,
        out_shape=jax.ShapeDtypeStruct((n0, 13), jnp.float32),
    )(h, params["out1"]["W"], params["out1"]["b"].reshape(1, -1),
      params["out2"]["W"], params["out2"]["b"].reshape(1, -1),
      params["out3"]["W"], params["out3"]["b"].reshape(1, -1))


def kernel(x, pos, params):
    f1, sel1 = _fps_pallas(pos, 2048, 64)
    f2, sel2 = _fps_pallas(sel1, 512, 16)
    return _forward(x, pos, params, (f1, f2), (sel1, sel2))


# speedup vs baseline: 1.0461x; 1.0461x over previous
import functools

import jax
import jax.numpy as jnp
from jax.experimental import pallas as pl
from jax.experimental.pallas import tpu as pltpu
from jax.experimental.pallas import tpu_sc as plsc

_K = 16


def _fps_body(n_samples, px_ref, py_ref, pz_ref, cx_ref, cy_ref, cz_ref,
              idx_ref, sx_ref, sy_ref, sz_ref, dist_ref):
    R, C = px_ref.shape
    iota = (jax.lax.broadcasted_iota(jnp.int32, (R, C), 0) * C
            + jax.lax.broadcasted_iota(jnp.int32, (R, C), 1))
    idx_ref[0:1, :] = jnp.zeros((1, 1), jnp.int32)
    sx_ref[0:1, :] = px_ref[0:1, 0:1]
    sy_ref[0:1, :] = py_ref[0:1, 0:1]
    sz_ref[0:1, :] = pz_ref[0:1, 0:1]
    dist_ref[...] = jnp.full((R, C), 1e18, jnp.float32)

    def body(i, carry):
        lx, ly, lz = carry
        dx = px_ref[...] - lx
        dy = py_ref[...] - ly
        dz = pz_ref[...] - lz
        d = dx * dx + dy * dy + dz * dz
        nd = jnp.minimum(dist_ref[...], d)
        dist_ref[...] = nd
        m = jnp.max(nd)
        cand = jnp.where(nd == m, iota, jnp.int32(2 ** 30))
        j = jnp.min(cand)
        idx_ref[pl.ds(i, 1), :] = jnp.reshape(j, (1, 1))
        nlx = cx_ref[j, 0]
        nly = cy_ref[j, 0]
        nlz = cz_ref[j, 0]
        sx_ref[pl.ds(i, 1), :] = jnp.reshape(nlx, (1, 1))
        sy_ref[pl.ds(i, 1), :] = jnp.reshape(nly, (1, 1))
        sz_ref[pl.ds(i, 1), :] = jnp.reshape(nlz, (1, 1))
        return nlx, nly, nlz

    jax.lax.fori_loop(1, n_samples, body,
                      (px_ref[0, 0], py_ref[0, 0], pz_ref[0, 0]))


def _fps_pallas(pos, n_samples, rows):
    n = pos.shape[0]
    cols = n // rows
    px = pos[:, 0].reshape(rows, cols)
    py = pos[:, 1].reshape(rows, cols)
    pz = pos[:, 2].reshape(rows, cols)
    out = pl.pallas_call(
        functools.partial(_fps_body, n_samples),
        out_shape=[jax.ShapeDtypeStruct((n_samples, 1), jnp.int32)]
        + [jax.ShapeDtypeStruct((n_samples, 1), jnp.float32)] * 3,
        scratch_shapes=[pltpu.VMEM((rows, cols), jnp.float32)],
    )(px, py, pz, pos[:, 0].reshape(n, 1), pos[:, 1].reshape(n, 1),
      pos[:, 2].reshape(n, 1))
    sel = jnp.concatenate([out[1], out[2], out[3]], axis=1)
    return out[0][:, 0], sel


def _knn_body(k, nb, self_exclude, q_ref, bt_ref, idx_ref, d2_ref):
    R = q_ref.shape[0]
    q = q_ref[...]
    bt = bt_ref[...]
    qq = jnp.sum(q * q, axis=1, keepdims=True)
    bb = jnp.sum(bt * bt, axis=0, keepdims=True)
    ab = jnp.dot(q, bt, preferred_element_type=jnp.float32)
    D = qq - 2.0 * ab + bb
    lane = jax.lax.broadcasted_iota(jnp.int32, (R, nb), 1)
    if self_exclude:
        row = (jax.lax.broadcasted_iota(jnp.int32, (R, nb), 0)
               + pl.program_id(0) * R)
        D = jnp.where(row == lane, 1e10, D)
    big = jnp.int32(2 ** 30)
    for e in range(k):
        m = jnp.min(D, axis=1, keepdims=True)
        j = jnp.min(jnp.where(D == m, lane, big), axis=1, keepdims=True)
        D = jnp.where(lane == j, 1e30, D)
        idx_ref[:, e:e + 1] = j
        d2_ref[:, e:e + 1] = jnp.maximum(m, 0.0)


def _knn_pallas(query, base, k, self_exclude=False, block_r=256):
    nq = query.shape[0]
    nb = base.shape[0]
    r = min(block_r, nq)
    bt = base.T
    idx, d2 = pl.pallas_call(
        functools.partial(_knn_body, k, nb, self_exclude),
        grid=(nq // r,),
        in_specs=[
            pl.BlockSpec((r, 3), lambda i: (i, 0)),
            pl.BlockSpec((3, nb), lambda i: (0, 0)),
        ],
        out_specs=[
            pl.BlockSpec((r, k), lambda i: (i, 0)),
            pl.BlockSpec((r, k), lambda i: (i, 0)),
        ],
        out_shape=[
            jax.ShapeDtypeStruct((nq, k), jnp.int32),
            jax.ShapeDtypeStruct((nq, k), jnp.float32),
        ],
    )(query, bt)
    return d2, idx


def _sc_gather_rows(table, idx):
    V, D = table.shape
    B = idx.shape[0]
    NW = 32
    C = B // (NW * 128)
    idx3 = idx.reshape(NW, C, 128)
    mesh = plsc.VectorSubcoreMesh(core_axis_name="c", subcore_axis_name="s")

    @functools.partial(
        pl.kernel, mesh=mesh,
        out_type=jax.ShapeDtypeStruct((B, D), jnp.float32),
        scratch_types=[
            pltpu.VMEM((128,), jnp.int32),
            pltpu.VMEM((128, D), jnp.float32),
            pltpu.SemaphoreType.DMA,
        ],
    )
    def k(table_hbm, idx_hbm, out_hbm, idx_v, rows_v, sem):
        wid = jax.lax.axis_index("s") * 2 + jax.lax.axis_index("c")

        def chunk(c, carry):
            pltpu.sync_copy(idx_hbm.at[wid, c], idx_v)
            pltpu.async_copy(table_hbm.at[idx_v], rows_v, sem).wait()
            base = (wid * C + c) * 128
            pltpu.sync_copy(rows_v, out_hbm.at[pl.ds(base, 128)])
            return carry

        jax.lax.fori_loop(0, C, chunk, 0)

    return k(table, idx3)


def _gather_rows(table, idx):
    v, d = table.shape
    b = idx.shape[0]
    dpad = (d + 127) // 128 * 128
    bpad = (b + 4095) // 4096 * 4096
    if dpad != d:
        table = jnp.pad(table, ((0, 0), (0, dpad - d)))
    if bpad != b:
        idx = jnp.pad(idx, (0, bpad - b))
    out = _sc_gather_rows(table, idx)
    return out[:b, :d]


def _mlp_bn_body(x_ref, w_ref, b_ref, g_ref, be_ref, o_ref):
    y = jnp.dot(x_ref[...], w_ref[...], preferred_element_type=jnp.float32)
    y = y + b_ref[...]
    mu = jnp.mean(y, axis=0, keepdims=True)
    var = jnp.mean((y - mu) * (y - mu), axis=0, keepdims=True)
    y = (y - mu) / jnp.sqrt(var + 1e-5) * g_ref[...] + be_ref[...]
    o_ref[...] = jnp.maximum(y, 0.0)


def _mlp_bn(p, x):
    n = x.shape[0]
    dout = p["W"].shape[1]
    return pl.pallas_call(
        _mlp_bn_body,
        out_shape=jax.ShapeDtypeStruct((n, dout), jnp.float32),
    )(x, p["W"], p["b"].reshape(1, -1), p["g"].reshape(1, -1),
      p["be"].reshape(1, -1))


def _maxpool_body(g_ref, o_ref):
    o_ref[...] = jnp.max(g_ref[...], axis=1)


def _maxpool_k(g3, bn=256):
    n, k, d = g3.shape
    bn = min(bn, n)
    return pl.pallas_call(
        _maxpool_body,
        grid=(n // bn,),
        in_specs=[pl.BlockSpec((bn, k, d), lambda i: (i, 0, 0))],
        out_specs=pl.BlockSpec((bn, d), lambda i: (i, 0)),
        out_shape=jax.ShapeDtypeStruct((n, d), jnp.float32),
    )(g3)


def _tf_pre_body(d, x_ref, pos_ref, wi_ref, bi_ref, wd_ref, bd_ref, ws_ref,
                 bs_ref, wv_ref, bv_ref, tab_ref, q_ref):
    h = jnp.dot(x_ref[...], wi_ref[...], preferred_element_type=jnp.float32)
    h = jnp.maximum(h + bi_ref[...], 0.0)
    q_ref[...] = jnp.dot(h, wd_ref[...],
                         preferred_element_type=jnp.float32) + bd_ref[...]
    tab_ref[...] = jnp.zeros(tab_ref.shape, jnp.float32)
    tab_ref[:, 0:d] = jnp.dot(h, ws_ref[...],
                              preferred_element_type=jnp.float32) + bs_ref[...]
    tab_ref[:, d:2 * d] = jnp.dot(h, wv_ref[...],
                                  preferred_element_type=jnp.float32) + bv_ref[...]
    tab_ref[:, 2 * d:2 * d + 3] = pos_ref[...]


def _tf_attn_body(d, g_ref, q_ref, pos_ref, w1_ref, b1_ref, w2_ref, b2_ref,
                  wa1_ref, ba1_ref, wa2_ref, ba2_ref, wo_ref, bo_ref, o_ref):
    bn, k, dp = g_ref.shape
    g3 = g_ref[...]
    pq3 = pos_ref[...].reshape(bn, 1, 3)
    rel3 = pq3 - g3[:, :, 2 * d:2 * d + 3]
    rel2 = rel3.reshape(bn * k, 3)
    t = jnp.dot(rel2, w1_ref[...], preferred_element_type=jnp.float32)
    t = jnp.maximum(t + b1_ref[...], 0.0)
    delta2 = jnp.dot(t, w2_ref[...],
                     preferred_element_type=jnp.float32) + b2_ref[...]
    delta3 = delta2.reshape(bn, k, d)
    q3 = q_ref[...].reshape(bn, 1, d)
    alpha3 = q3 - g3[:, :, 0:d] + delta3
    a2 = alpha3.reshape(bn * k, d)
    t2 = jnp.dot(a2, wa1_ref[...], preferred_element_type=jnp.float32)
    t2 = jnp.maximum(t2 + ba1_ref[...], 0.0)
    a2 = jnp.dot(t2, wa2_ref[...],
                 preferred_element_type=jnp.float32) + ba2_ref[...]
    alpha3 = a2.reshape(bn, k, d)
    m = jnp.max(alpha3, axis=1, keepdims=True)
    e = jnp.exp(alpha3 - m)
    p = e / jnp.sum(e, axis=1, keepdims=True)
    out = jnp.sum(p * (g3[:, :, d:2 * d] + delta3), axis=1)
    y = jnp.dot(out, wo_ref[...], preferred_element_type=jnp.float32)
    o_ref[...] = jnp.maximum(y + bo_ref[...], 0.0)


def _transformer_block(p, x, pos, nbr):
    n, d = x.shape
    dpad = (2 * d + 3 + 127) // 128 * 128
    tab, q = pl.pallas_call(
        functools.partial(_tf_pre_body, d),
        out_shape=[
            jax.ShapeDtypeStruct((n, dpad), jnp.float32),
            jax.ShapeDtypeStruct((n, d), jnp.float32),
        ],
    )(x, pos, p["lin_in"]["W"], p["lin_in"]["b"].reshape(1, -1),
      p["lin_dst"]["W"], p["lin_dst"]["b"].reshape(1, -1),
      p["lin_src"]["W"], p["lin_src"]["b"].reshape(1, -1),
      p["lin"]["W"], p["lin"]["b"].reshape(1, -1))
    g = _gather_rows(tab, nbr.reshape(-1))
    g3 = g.reshape(n, _K, dpad)
    bn = 128
    return pl.pallas_call(
        functools.partial(_tf_attn_body, d),
        grid=(n // bn,),
        in_specs=[
            pl.BlockSpec((bn, _K, dpad), lambda i: (i, 0, 0)),
            pl.BlockSpec((bn, d), lambda i: (i, 0)),
            pl.BlockSpec((bn, 3), lambda i: (i, 0)),
        ] + [pl.BlockSpec(None, lambda i: (0, 0))] * 10,
        out_specs=pl.BlockSpec((bn, d), lambda i: (i, 0)),
        out_shape=jax.ShapeDtypeStruct((n, d), jnp.float32),
    )(g3, q, pos,
      p["pos1"]["W"], p["pos1"]["b"].reshape(1, -1),
      p["pos2"]["W"], p["pos2"]["b"].reshape(1, -1),
      p["attn1"]["W"], p["attn1"]["b"].reshape(1, -1),
      p["attn2"]["W"], p["attn2"]["b"].reshape(1, -1),
      p["lin_out"]["W"], p["lin_out"]["b"].reshape(1, -1))


def _interp_body(x_ref, d2_ref, f0_ref, f1_ref, f2_ref, o_ref):
    w = 1.0 / jnp.maximum(d2_ref[...], 1e-16)
    w = w / jnp.sum(w, axis=1, keepdims=True)
    o_ref[...] = (x_ref[...] + w[:, 0:1] * f0_ref[...]
                  + w[:, 1:2] * f1_ref[...] + w[:, 2:3] * f2_ref[...])


def _interp_add(x_mlp, feat, d2, idx):
    n = x_mlp.shape[0]
    d = feat.shape[1]
    f = [_gather_rows(feat, idx[:, j]) for j in range(3)]
    return pl.pallas_call(
        _interp_body,
        out_shape=jax.ShapeDtypeStruct((n, d), jnp.float32),
    )(x_mlp, d2, f[0], f[1], f[2])


def _head_body(h_ref, w1_ref, b1_ref, w2_ref, b2_ref, w3_ref, b3_ref, o_ref):
    o = jnp.dot(h_ref[...], w1_ref[...], preferred_element_type=jnp.float32)
    o = jnp.maximum(o + b1_ref[...], 0.0)
    o = jnp.dot(o, w2_ref[...], preferred_element_type=jnp.float32)
    o = jnp.maximum(o + b2_ref[...], 0.0)
    o = jnp.dot(o, w3_ref[...], preferred_element_type=jnp.float32)
    o_ref[...] = o + b3_ref[...]


def _summit_body(h_ref, w_ref, b_ref, o_ref):
    o = jnp.dot(h_ref[...], w_ref[...], preferred_element_type=jnp.float32)
    o_ref[...] = jnp.maximum(o + b_ref[...], 0.0)


def _forward(x, pos, params, fps_idx, sub_pos):
    h = _mlp_bn(params["mlp_input"], x)
    out_x = [h]
    out_pos = [pos]
    for i in range(2):
        p_prev = out_pos[-1]
        sp = sub_pos[i]
        _, nbr = _knn_pallas(sp, p_prev, _K)
        hh = _mlp_bn(params["td"][i], out_x[-1])
        gh = _gather_rows(hh, nbr.reshape(-1))
        h = _maxpool_k(gh.reshape(sp.shape[0], _K, hh.shape[1]))
        nbr_g = _knn_pallas(sp, sp, _K, self_exclude=True)[1]
        h = _transformer_block(params["tf_down"][i], h, sp, nbr_g)
        out_x.append(h)
        out_pos.append(sp)
    n2, d2dim = h.shape
    h = pl.pallas_call(
        _summit_body,
        out_shape=jax.ShapeDtypeStruct((n2, d2dim), jnp.float32),
    )(h, params["mlp_summit"]["W"], params["mlp_summit"]["b"].reshape(1, -1))
    h = _transformer_block(params["tf_summit"], h, out_pos[-1],
                           _knn_pallas(out_pos[-1], out_pos[-1], _K,
                                       self_exclude=True)[1])
    for i in range(2):
        pu = params["tu"][-i - 1]
        x_skip = out_x[-i - 2]
        pos_up = out_pos[-i - 2]
        pos_sub = out_pos[-i - 1]
        h_sub = _mlp_bn(pu["mlp_sub"], h)
        d2i, idxi = _knn_pallas(pos_up, pos_sub, 3)
        x_mlp = _mlp_bn(pu["mlp"], x_skip)
        h = _interp_add(x_mlp, h_sub, d2i, idxi)
        h = _transformer_block(params["tf_up"][-i - 1], h, pos_up,
                               _knn_pallas(pos_up, pos_up, _K,
                                           self_exclude=True)[1])
    n0 = h.shape[0]
    return pl.pallas_call(
        _head_body,
        out_shape=jax.ShapeDtypeStruct((n0, 13), jnp.float32),
    )(h, params["out1"]["W"], params["out1"]["b"].reshape(1, -1),
      params["out2"]["W"], params["out2"]["b"].reshape(1, -1),
      params["out3"]["W"], params["out3"]["b"].reshape(1, -1))


def kernel(x, pos, params):
    f1, sel1 = _fps_pallas(pos, 2048, 64)
    f2, sel2 = _fps_pallas(sel1, 512, 16)
    return _forward(x, pos, params, (f1, f2), (sel1, sel2))
